# baseline (speedup 1.0000x reference)
import jax
import jax.numpy as jnp
from jax import lax
from jax.experimental import pallas as pl
from jax.experimental.pallas import tpu as pltpu

B, T, J, DIM, HEADS = 64, 64, 25, 128, 4
CH = DIM // HEADS
E = 64
GB = 8
R = GB * J
G = B * T
N = G * J


def _gat_body(x_ref, ei_ref, w_ref, atts_ref, attd_ref, bias_ref, o_ref):
    f32 = jnp.float32
    i32 = jnp.int32

    es = ei_ref[0, 0:1, :]
    ed = ei_ref[0, 1:2, :]
    Hd = (lax.broadcasted_iota(i32, (J, E), 0) == ed).astype(f32)
    Hs = (lax.broadcasted_iota(i32, (J, E), 0) == es).astype(f32)
    C = lax.dot_general(Hd, Hs, (((1,), (1,)), ((), ())),
                        preferred_element_type=f32)
    eye = (lax.broadcasted_iota(i32, (J, J), 0)
           == lax.broadcasted_iota(i32, (J, J), 1)).astype(f32)
    C = C + eye
    logC = jnp.where(C > 0.0, jnp.log(C), -1e30)

    U = ((lax.broadcasted_iota(i32, (R, J), 0) % J)
         == lax.broadcasted_iota(i32, (R, J), 1)).astype(f32)
    Lg = jnp.dot(U, logC, preferred_element_type=f32)
    Lfull = lax.dot_general(Lg, U, (((1,), (1,)), ((), ())),
                            preferred_element_type=f32)
    rg = lax.broadcasted_iota(i32, (R, R), 0) // J
    cg = lax.broadcasted_iota(i32, (R, R), 1) // J
    Lfull = jnp.where(rg == cg, Lfull, -1e30)

    x2 = x_ref[:].reshape(R, DIM)
    xh = jnp.dot(x2, w_ref[:], preferred_element_type=f32)

    k2 = lax.broadcasted_iota(i32, (DIM, 2 * HEADS), 0) // CH
    c2 = lax.broadcasted_iota(i32, (DIM, 2 * HEADS), 1)
    M = (jnp.where(k2 == c2, atts_ref[:], 0.0)
         + jnp.where(k2 == c2 - HEADS, attd_ref[:], 0.0))
    Acol = jnp.dot(xh, M, preferred_element_type=f32)
    Arow = lax.dot_general(M, xh, (((0,), (1,)), ((), ())),
                           preferred_element_type=f32)

    ones_col = jnp.ones((R, 1), f32)
    outs = []
    for h in range(HEADS):
        S = Acol[:, HEADS + h:HEADS + h + 1] + Arow[h:h + 1, :]
        S = jnp.maximum(S, 0.2 * S) + Lfull
        ex = jnp.exp(S)
        xe = jnp.concatenate([xh[:, h * CH:(h + 1) * CH], ones_col], axis=1)
        u = jnp.dot(ex, xe, preferred_element_type=f32)
        recip = 1.0 / (u[:, CH:CH + 1] + 1e-16)
        outs.append(u[:, :CH] * recip)
    res = jnp.concatenate(outs, axis=1) + bias_ref[:]
    o_ref[...] = res.reshape(o_ref.shape)


def kernel(x, edge_index, W, att_src, att_dst, bias):
    ei3 = edge_index.reshape(1, 2, E)
    atts = att_src.reshape(DIM, 1)
    attd = att_dst.reshape(DIM, 1)
    bias2 = bias.reshape(1, DIM)

    out = pl.pallas_call(
        _gat_body,
        grid=(B, T // GB),
        in_specs=[
            pl.BlockSpec((1, GB, J, DIM), lambda b, t: (b, t, 0, 0)),
            pl.BlockSpec((1, 2, E), lambda b, t: (0, 0, 0)),
            pl.BlockSpec((DIM, DIM), lambda b, t: (0, 0)),
            pl.BlockSpec((DIM, 1), lambda b, t: (0, 0)),
            pl.BlockSpec((DIM, 1), lambda b, t: (0, 0)),
            pl.BlockSpec((1, DIM), lambda b, t: (0, 0)),
        ],
        out_specs=pl.BlockSpec((1, GB, J, DIM), lambda b, t: (b, t, 0, 0)),
        out_shape=jax.ShapeDtypeStruct((B, T, J, DIM), jnp.float32),
        compiler_params=pltpu.CompilerParams(
            dimension_semantics=("parallel", "parallel")),
    )(x, ei3, W, atts, attd, bias2)
    return out


# speedup vs baseline: 1.2360x; 1.2360x over previous
import jax
import jax.numpy as jnp
from jax import lax
from jax.experimental import pallas as pl
from jax.experimental.pallas import tpu as pltpu

B, T, J, DIM, HEADS = 64, 64, 25, 128, 4
CH = DIM // HEADS
E = 64
GB = 8
R = GB * J
TPP = 2
G = B * T
N = G * J


def _gat_body(x_ref, ei_ref, w_ref, atts_ref, attd_ref, bias_ref, o_ref):
    f32 = jnp.float32
    i32 = jnp.int32

    es = ei_ref[0, 0:1, :]
    ed = ei_ref[0, 1:2, :]
    Hd = (lax.broadcasted_iota(i32, (J, E), 0) == ed).astype(f32)
    Hs = (lax.broadcasted_iota(i32, (J, E), 0) == es).astype(f32)
    C = lax.dot_general(Hd, Hs, (((1,), (1,)), ((), ())),
                        preferred_element_type=f32)
    eye = (lax.broadcasted_iota(i32, (J, J), 0)
           == lax.broadcasted_iota(i32, (J, J), 1)).astype(f32)
    C = C + eye
    logC = jnp.where(C > 0.0, jnp.log(C), -1e30)

    U = ((lax.broadcasted_iota(i32, (R, J), 0) % J)
         == lax.broadcasted_iota(i32, (R, J), 1)).astype(f32)
    Lg = jnp.dot(U, logC, preferred_element_type=f32)
    Lfull = lax.dot_general(Lg, U, (((1,), (1,)), ((), ())),
                            preferred_element_type=f32)
    rg = lax.broadcasted_iota(i32, (R, R), 0) // J
    cg = lax.broadcasted_iota(i32, (R, R), 1) // J
    Lfull = jnp.where(rg == cg, Lfull, -1e30)

    k2 = lax.broadcasted_iota(i32, (DIM, 2 * HEADS), 0) // CH
    c2 = lax.broadcasted_iota(i32, (DIM, 2 * HEADS), 1)
    M = (jnp.where(k2 == c2, atts_ref[:], 0.0)
         + jnp.where(k2 == c2 - HEADS, attd_ref[:], 0.0))

    ones_col = jnp.ones((R, 1), f32)
    for t in range(TPP):
        x2 = x_ref[0, t * GB:(t + 1) * GB].reshape(R, DIM)
        xh = jnp.dot(x2, w_ref[:], preferred_element_type=f32)
        Acol = jnp.dot(xh, M, preferred_element_type=f32)
        Arow = lax.dot_general(M, xh, (((0,), (1,)), ((), ())),
                               preferred_element_type=f32)
        outs = []
        for h in range(HEADS):
            S = Acol[:, HEADS + h:HEADS + h + 1] + Arow[h:h + 1, :]
            S = jnp.maximum(S, 0.2 * S) + Lfull
            ex = jnp.exp(S)
            xe = jnp.concatenate([xh[:, h * CH:(h + 1) * CH], ones_col],
                                 axis=1)
            u = jnp.dot(ex, xe, preferred_element_type=f32)
            recip = 1.0 / (u[:, CH:CH + 1] + 1e-16)
            outs.append(u[:, :CH] * recip)
        res = jnp.concatenate(outs, axis=1) + bias_ref[:]
        o_ref[0, t * GB:(t + 1) * GB] = res.reshape(GB, J, DIM)


def kernel(x, edge_index, W, att_src, att_dst, bias):
    ei3 = edge_index.reshape(1, 2, E)
    atts = att_src.reshape(DIM, 1)
    attd = att_dst.reshape(DIM, 1)
    bias2 = bias.reshape(1, DIM)

    out = pl.pallas_call(
        _gat_body,
        grid=(B, T // (TPP * GB)),
        in_specs=[
            pl.BlockSpec((1, TPP * GB, J, DIM), lambda b, t: (b, t, 0, 0)),
            pl.BlockSpec((1, 2, E), lambda b, t: (0, 0, 0)),
            pl.BlockSpec((DIM, DIM), lambda b, t: (0, 0)),
            pl.BlockSpec((DIM, 1), lambda b, t: (0, 0)),
            pl.BlockSpec((DIM, 1), lambda b, t: (0, 0)),
            pl.BlockSpec((1, DIM), lambda b, t: (0, 0)),
        ],
        out_specs=pl.BlockSpec((1, TPP * GB, J, DIM), lambda b, t: (b, t, 0, 0)),
        out_shape=jax.ShapeDtypeStruct((B, T, J, DIM), jnp.float32),
        compiler_params=pltpu.CompilerParams(
            dimension_semantics=("parallel", "parallel")),
    )(x, ei3, W, atts, attd, bias2)
    return out


# speedup vs baseline: 1.3655x; 1.1047x over previous
import jax
import jax.numpy as jnp
from jax import lax
from jax.experimental import pallas as pl
from jax.experimental.pallas import tpu as pltpu

B, T, J, DIM, HEADS = 64, 64, 25, 128, 4
CH = DIM // HEADS
E = 64
GB = 8
R = GB * J
TPP = 4
G = B * T
N = G * J


def _gat_body(x_ref, ei_ref, w_ref, atts_ref, attd_ref, bias_ref, o_ref):
    f32 = jnp.float32
    i32 = jnp.int32

    es = ei_ref[0, 0:1, :]
    ed = ei_ref[0, 1:2, :]
    Hd = (lax.broadcasted_iota(i32, (J, E), 0) == ed).astype(f32)
    Hs = (lax.broadcasted_iota(i32, (J, E), 0) == es).astype(f32)
    C = lax.dot_general(Hd, Hs, (((1,), (1,)), ((), ())),
                        preferred_element_type=f32)
    eye = (lax.broadcasted_iota(i32, (J, J), 0)
           == lax.broadcasted_iota(i32, (J, J), 1)).astype(f32)
    C = C + eye
    logC = jnp.where(C > 0.0, jnp.log(C), -1e30)

    U = ((lax.broadcasted_iota(i32, (R, J), 0) % J)
         == lax.broadcasted_iota(i32, (R, J), 1)).astype(f32)
    Lg = jnp.dot(U, logC, preferred_element_type=f32)
    Lfull = lax.dot_general(Lg, U, (((1,), (1,)), ((), ())),
                            preferred_element_type=f32)
    rg = lax.broadcasted_iota(i32, (R, R), 0) // J
    cg = lax.broadcasted_iota(i32, (R, R), 1) // J
    Lfull = jnp.where(rg == cg, Lfull, -1e30)

    k2 = lax.broadcasted_iota(i32, (DIM, 2 * HEADS), 0) // CH
    c2 = lax.broadcasted_iota(i32, (DIM, 2 * HEADS), 1)
    M = (jnp.where(k2 == c2, atts_ref[:], 0.0)
         + jnp.where(k2 == c2 - HEADS, attd_ref[:], 0.0))

    ones_col = jnp.ones((R, 1), f32)
    for t in range(TPP):
        x2 = x_ref[0, t * GB:(t + 1) * GB].reshape(R, DIM)
        xh = jnp.dot(x2, w_ref[:], preferred_element_type=f32)
        Acol = jnp.dot(xh, M, preferred_element_type=f32)
        Arow = lax.dot_general(M, xh, (((0,), (1,)), ((), ())),
                               preferred_element_type=f32)
        outs = []
        for h in range(HEADS):
            S = Acol[:, HEADS + h:HEADS + h + 1] + Arow[h:h + 1, :]
            S = jnp.maximum(S, 0.2 * S) + Lfull
            ex = jnp.exp(S)
            xe = jnp.concatenate([xh[:, h * CH:(h + 1) * CH], ones_col],
                                 axis=1)
            u = jnp.dot(ex, xe, preferred_element_type=f32)
            recip = 1.0 / (u[:, CH:CH + 1] + 1e-16)
            outs.append(u[:, :CH] * recip)
        res = jnp.concatenate(outs, axis=1) + bias_ref[:]
        o_ref[0, t * GB:(t + 1) * GB] = res.reshape(GB, J, DIM)


def kernel(x, edge_index, W, att_src, att_dst, bias):
    ei3 = edge_index.reshape(1, 2, E)
    atts = att_src.reshape(DIM, 1)
    attd = att_dst.reshape(DIM, 1)
    bias2 = bias.reshape(1, DIM)

    out = pl.pallas_call(
        _gat_body,
        grid=(B, T // (TPP * GB)),
        in_specs=[
            pl.BlockSpec((1, TPP * GB, J, DIM), lambda b, t: (b, t, 0, 0)),
            pl.BlockSpec((1, 2, E), lambda b, t: (0, 0, 0)),
            pl.BlockSpec((DIM, DIM), lambda b, t: (0, 0)),
            pl.BlockSpec((DIM, 1), lambda b, t: (0, 0)),
            pl.BlockSpec((DIM, 1), lambda b, t: (0, 0)),
            pl.BlockSpec((1, DIM), lambda b, t: (0, 0)),
        ],
        out_specs=pl.BlockSpec((1, TPP * GB, J, DIM), lambda b, t: (b, t, 0, 0)),
        out_shape=jax.ShapeDtypeStruct((B, T, J, DIM), jnp.float32),
        compiler_params=pltpu.CompilerParams(
            dimension_semantics=("parallel", "parallel")),
    )(x, ei3, W, atts, attd, bias2)
    return out


# speedup vs baseline: 1.4337x; 1.0500x over previous
import jax
import jax.numpy as jnp
from jax import lax
from jax.experimental import pallas as pl
from jax.experimental.pallas import tpu as pltpu

B, T, J, DIM, HEADS = 64, 64, 25, 128, 4
CH = DIM // HEADS
E = 64
GB = 8
R = GB * J
TPP = 8
G = B * T
N = G * J


def _gat_body(x_ref, ei_ref, w_ref, atts_ref, attd_ref, bias_ref, o_ref):
    f32 = jnp.float32
    i32 = jnp.int32

    es = ei_ref[0, 0:1, :]
    ed = ei_ref[0, 1:2, :]
    Hd = (lax.broadcasted_iota(i32, (J, E), 0) == ed).astype(f32)
    Hs = (lax.broadcasted_iota(i32, (J, E), 0) == es).astype(f32)
    C = lax.dot_general(Hd, Hs, (((1,), (1,)), ((), ())),
                        preferred_element_type=f32)
    eye = (lax.broadcasted_iota(i32, (J, J), 0)
           == lax.broadcasted_iota(i32, (J, J), 1)).astype(f32)
    C = C + eye
    logC = jnp.where(C > 0.0, jnp.log(C), -1e30)

    U = ((lax.broadcasted_iota(i32, (R, J), 0) % J)
         == lax.broadcasted_iota(i32, (R, J), 1)).astype(f32)
    Lg = jnp.dot(U, logC, preferred_element_type=f32)
    Lfull = lax.dot_general(Lg, U, (((1,), (1,)), ((), ())),
                            preferred_element_type=f32)
    rg = lax.broadcasted_iota(i32, (R, R), 0) // J
    cg = lax.broadcasted_iota(i32, (R, R), 1) // J
    Lfull = jnp.where(rg == cg, Lfull, -1e30)

    k2 = lax.broadcasted_iota(i32, (DIM, 2 * HEADS), 0) // CH
    c2 = lax.broadcasted_iota(i32, (DIM, 2 * HEADS), 1)
    M = (jnp.where(k2 == c2, atts_ref[:], 0.0)
         + jnp.where(k2 == c2 - HEADS, attd_ref[:], 0.0))

    ones_col = jnp.ones((R, 1), f32)
    for t in range(TPP):
        x2 = x_ref[0, t * GB:(t + 1) * GB].reshape(R, DIM)
        xh = jnp.dot(x2, w_ref[:], preferred_element_type=f32)
        Acol = jnp.dot(xh, M, preferred_element_type=f32)
        Arow = lax.dot_general(M, xh, (((0,), (1,)), ((), ())),
                               preferred_element_type=f32)
        outs = []
        for h in range(HEADS):
            S = Acol[:, HEADS + h:HEADS + h + 1] + Arow[h:h + 1, :]
            S = jnp.maximum(S, 0.2 * S) + Lfull
            ex = jnp.exp(S)
            xe = jnp.concatenate([xh[:, h * CH:(h + 1) * CH], ones_col],
                                 axis=1)
            u = jnp.dot(ex, xe, preferred_element_type=f32)
            recip = 1.0 / (u[:, CH:CH + 1] + 1e-16)
            outs.append(u[:, :CH] * recip)
        res = jnp.concatenate(outs, axis=1) + bias_ref[:]
        o_ref[0, t * GB:(t + 1) * GB] = res.reshape(GB, J, DIM)


def kernel(x, edge_index, W, att_src, att_dst, bias):
    ei3 = edge_index.reshape(1, 2, E)
    atts = att_src.reshape(DIM, 1)
    attd = att_dst.reshape(DIM, 1)
    bias2 = bias.reshape(1, DIM)

    out = pl.pallas_call(
        _gat_body,
        grid=(B, T // (TPP * GB)),
        in_specs=[
            pl.BlockSpec((1, TPP * GB, J, DIM), lambda b, t: (b, t, 0, 0)),
            pl.BlockSpec((1, 2, E), lambda b, t: (0, 0, 0)),
            pl.BlockSpec((DIM, DIM), lambda b, t: (0, 0)),
            pl.BlockSpec((DIM, 1), lambda b, t: (0, 0)),
            pl.BlockSpec((DIM, 1), lambda b, t: (0, 0)),
            pl.BlockSpec((1, DIM), lambda b, t: (0, 0)),
        ],
        out_specs=pl.BlockSpec((1, TPP * GB, J, DIM), lambda b, t: (b, t, 0, 0)),
        out_shape=jax.ShapeDtypeStruct((B, T, J, DIM), jnp.float32),
        compiler_params=pltpu.CompilerParams(
            dimension_semantics=("parallel", "parallel")),
    )(x, ei3, W, atts, attd, bias2)
    return out


# speedup vs baseline: 1.4812x; 1.0331x over previous
import jax
import jax.numpy as jnp
from jax import lax
from jax.experimental import pallas as pl
from jax.experimental.pallas import tpu as pltpu

B, T, J, DIM, HEADS = 64, 64, 25, 128, 4
CH = DIM // HEADS
E = 64
GB = 8
R = GB * J
TPP = 8
G = B * T
N = G * J


def _gat_body(x_ref, ei_ref, w_ref, atts_ref, attd_ref, bias_ref, o_ref):
    f32 = jnp.float32
    i32 = jnp.int32

    es = ei_ref[0, 0:1, :]
    ed = ei_ref[0, 1:2, :]
    Hd = (lax.broadcasted_iota(i32, (J, E), 0) == ed).astype(f32)
    Hs = (lax.broadcasted_iota(i32, (J, E), 0) == es).astype(f32)
    C = lax.dot_general(Hd, Hs, (((1,), (1,)), ((), ())),
                        preferred_element_type=f32)
    eye = (lax.broadcasted_iota(i32, (J, J), 0)
           == lax.broadcasted_iota(i32, (J, J), 1)).astype(f32)
    C = C + eye
    logC = jnp.where(C > 0.0, jnp.log2(C), -1e30)

    U = ((lax.broadcasted_iota(i32, (R, J), 0) % J)
         == lax.broadcasted_iota(i32, (R, J), 1)).astype(f32)
    Lg = jnp.dot(U, logC, preferred_element_type=f32)
    Lfull = lax.dot_general(Lg, U, (((1,), (1,)), ((), ())),
                            preferred_element_type=f32)
    rg = lax.broadcasted_iota(i32, (R, R), 0) // J
    cg = lax.broadcasted_iota(i32, (R, R), 1) // J
    Lfull = jnp.where(rg == cg, Lfull, -1e30)

    LOG2E = 1.4426950408889634
    k2 = lax.broadcasted_iota(i32, (DIM, 2 * HEADS), 0) // CH
    c2 = lax.broadcasted_iota(i32, (DIM, 2 * HEADS), 1)
    M = (jnp.where(k2 == c2, atts_ref[:], 0.0)
         + jnp.where(k2 == c2 - HEADS, attd_ref[:], 0.0)) * LOG2E

    ones_col = jnp.ones((R, 1), f32)
    for t in range(TPP):
        x2 = x_ref[0, t * GB:(t + 1) * GB].reshape(R, DIM)
        xh = jnp.dot(x2, w_ref[:], preferred_element_type=f32)
        Acol = jnp.dot(xh, M, preferred_element_type=f32)
        Arow = lax.dot_general(M, xh, (((0,), (1,)), ((), ())),
                               preferred_element_type=f32)
        outs = []
        for h in range(HEADS):
            S = Acol[:, HEADS + h:HEADS + h + 1] + Arow[h:h + 1, :]
            S = jnp.maximum(S, 0.2 * S) + Lfull
            ex = jnp.exp2(S)
            xe = jnp.concatenate([xh[:, h * CH:(h + 1) * CH], ones_col],
                                 axis=1)
            u = jnp.dot(ex, xe, preferred_element_type=f32)
            recip = 1.0 / (u[:, CH:CH + 1] + 1e-16)
            outs.append(u[:, :CH] * recip)
        res = jnp.concatenate(outs, axis=1) + bias_ref[:]
        o_ref[0, t * GB:(t + 1) * GB] = res.reshape(GB, J, DIM)


def kernel(x, edge_index, W, att_src, att_dst, bias):
    ei3 = edge_index.reshape(1, 2, E)
    atts = att_src.reshape(DIM, 1)
    attd = att_dst.reshape(DIM, 1)
    bias2 = bias.reshape(1, DIM)

    out = pl.pallas_call(
        _gat_body,
        grid=(B, T // (TPP * GB)),
        in_specs=[
            pl.BlockSpec((1, TPP * GB, J, DIM), lambda b, t: (b, t, 0, 0)),
            pl.BlockSpec((1, 2, E), lambda b, t: (0, 0, 0)),
            pl.BlockSpec((DIM, DIM), lambda b, t: (0, 0)),
            pl.BlockSpec((DIM, 1), lambda b, t: (0, 0)),
            pl.BlockSpec((DIM, 1), lambda b, t: (0, 0)),
            pl.BlockSpec((1, DIM), lambda b, t: (0, 0)),
        ],
        out_specs=pl.BlockSpec((1, TPP * GB, J, DIM), lambda b, t: (b, t, 0, 0)),
        out_shape=jax.ShapeDtypeStruct((B, T, J, DIM), jnp.float32),
        compiler_params=pltpu.CompilerParams(
            dimension_semantics=("parallel", "parallel")),
    )(x, ei3, W, atts, attd, bias2)
    return out


# speedup vs baseline: 6.2727x; 4.2350x over previous
import jax
import jax.numpy as jnp
from jax import lax
from jax.experimental import pallas as pl
from jax.experimental.pallas import tpu as pltpu

B, T, J, DIM, HEADS = 64, 64, 25, 128, 4
CH = DIM // HEADS
E = 64
GB = 8
R = GB * J
TPP = 16
G = B * T
N = G * J


def _gat_body(x_ref, ei_ref, w_ref, atts_ref, attd_ref, bias_ref, o_ref):
    f32 = jnp.float32
    i32 = jnp.int32

    es = ei_ref[0, 0:1, :]
    ed = ei_ref[0, 1:2, :]
    Hd = (lax.broadcasted_iota(i32, (J, E), 0) == ed).astype(f32)
    Hs = (lax.broadcasted_iota(i32, (J, E), 0) == es).astype(f32)
    C = lax.dot_general(Hd, Hs, (((1,), (1,)), ((), ())),
                        preferred_element_type=f32)
    eye = (lax.broadcasted_iota(i32, (J, J), 0)
           == lax.broadcasted_iota(i32, (J, J), 1)).astype(f32)
    C = C + eye
    logC = jnp.where(C > 0.0, jnp.log2(C), -1e30)

    U = ((lax.broadcasted_iota(i32, (R, J), 0) % J)
         == lax.broadcasted_iota(i32, (R, J), 1)).astype(f32)
    Lg = jnp.dot(U, logC, preferred_element_type=f32)
    Lfull = lax.dot_general(Lg, U, (((1,), (1,)), ((), ())),
                            preferred_element_type=f32)
    rg = lax.broadcasted_iota(i32, (R, R), 0) // J
    cg = lax.broadcasted_iota(i32, (R, R), 1) // J
    Lfull = jnp.where(rg == cg, Lfull, -1e30)

    LOG2E = 1.4426950408889634
    k2 = lax.broadcasted_iota(i32, (DIM, 2 * HEADS), 0) // CH
    c2 = lax.broadcasted_iota(i32, (DIM, 2 * HEADS), 1)
    M = (jnp.where(k2 == c2, atts_ref[:], 0.0)
         + jnp.where(k2 == c2 - HEADS, attd_ref[:], 0.0)) * LOG2E

    ones_col = jnp.ones((R, 1), f32)
    for t in range(TPP):
        x2 = x_ref[0, t * GB:(t + 1) * GB].reshape(R, DIM)
        xh = jnp.dot(x2, w_ref[:], preferred_element_type=f32)
        Acol = jnp.dot(xh, M, preferred_element_type=f32)
        Arow = lax.dot_general(M, xh, (((0,), (1,)), ((), ())),
                               preferred_element_type=f32)
        outs = []
        for h in range(HEADS):
            S = Acol[:, HEADS + h:HEADS + h + 1] + Arow[h:h + 1, :]
            S = jnp.maximum(S, 0.2 * S) + Lfull
            ex = jnp.exp2(S)
            xe = jnp.concatenate([xh[:, h * CH:(h + 1) * CH], ones_col],
                                 axis=1)
            u = jnp.dot(ex, xe, preferred_element_type=f32)
            recip = 1.0 / (u[:, CH:CH + 1] + 1e-16)
            outs.append(u[:, :CH] * recip)
        res = jnp.concatenate(outs, axis=1) + bias_ref[:]
        o_ref[0, t * GB:(t + 1) * GB] = res.reshape(GB, J, DIM)


def kernel(x, edge_index, W, att_src, att_dst, bias):
    ei3 = edge_index.reshape(1, 2, E)
    atts = att_src.reshape(DIM, 1)
    attd = att_dst.reshape(DIM, 1)
    bias2 = bias.reshape(1, DIM)

    out = pl.pallas_call(
        _gat_body,
        grid=(B, T // (TPP * GB)),
        in_specs=[
            pl.BlockSpec((1, TPP * GB, J, DIM), lambda b, t: (b, t, 0, 0)),
            pl.BlockSpec((1, 2, E), lambda b, t: (0, 0, 0)),
            pl.BlockSpec((DIM, DIM), lambda b, t: (0, 0)),
            pl.BlockSpec((DIM, 1), lambda b, t: (0, 0)),
            pl.BlockSpec((DIM, 1), lambda b, t: (0, 0)),
            pl.BlockSpec((1, DIM), lambda b, t: (0, 0)),
        ],
        out_specs=pl.BlockSpec((1, TPP * GB, J, DIM), lambda b, t: (b, t, 0, 0)),
        out_shape=jax.ShapeDtypeStruct((B, T, J, DIM), jnp.float32),
        compiler_params=pltpu.CompilerParams(
            dimension_semantics=("parallel", "parallel")),
    )(x, ei3, W, atts, attd, bias2)
    return out
